# baseline (speedup 1.0000x reference)
import functools

import jax
import jax.numpy as jnp
from jax import lax
from jax.experimental import pallas as pl
from jax.experimental.pallas import tpu as pltpu
from jax.experimental.pallas import tpu_sc as plsc

_NUM_CONT = 13
_NUM_CAT = 26
_NUM_TOK = _NUM_CONT + _NUM_CAT
_NUM_BINS = 50
_VOCAB = 100000
_EMB = 64
_BATCH = 4096

_NC = 2
_NS = 16
_NW = _NC * _NS
_BPW = _BATCH // _NW
_ROWS = _BPW * _NUM_CONT
_CAT_HALF = _BPW // 2
_GCHUNK = 128
_NCHUNK = _ROWS // _GCHUNK
_NVEC = _ROWS // 16


def _sc_gather_tokens(cont_flat, cat_flat, bin_tab, cat_tab):
    mesh = plsc.VectorSubcoreMesh(core_axis_name="c", subcore_axis_name="s")

    @functools.partial(
        pl.kernel,
        out_type=jax.ShapeDtypeStruct((_NUM_TOK * _BATCH, _EMB), jnp.float32),
        mesh=mesh,
        scratch_types=[
            pltpu.VMEM((_ROWS,), jnp.float32),
            pltpu.VMEM((_ROWS,), jnp.int32),
            pltpu.VMEM((_ROWS,), jnp.int32),
            pltpu.VMEM((_NCHUNK, _GCHUNK), jnp.int32),
            pltpu.VMEM((_ROWS, _EMB), jnp.float32),
            pltpu.SemaphoreType.DMA,
        ],
        compiler_params=pltpu.CompilerParams(
            needs_layout_passes=False, use_tc_tiling_on_sc=False
        ),
    )
    def gather_kernel(cont_hbm, cat_hbm, bin_hbm, ctab_hbm, out_hbm,
                      feat_v, cati_v, idx_v, didx_v, rows_v, sem):
        wid = lax.axis_index("s") * _NC + lax.axis_index("c")
        base_b = wid * _BPW

        def move_rows(table_hbm):
            def g_body(j, _):
                pltpu.async_copy(
                    table_hbm.at[idx_v.at[pl.ds(j * _GCHUNK, _GCHUNK)]],
                    rows_v.at[pl.ds(j * _GCHUNK, _GCHUNK)],
                    sem,
                ).wait()
                pltpu.async_copy(
                    rows_v.at[pl.ds(j * _GCHUNK, _GCHUNK)],
                    out_hbm.at[didx_v.at[j]],
                    sem,
                ).wait()
                return 0

            lax.fori_loop(0, _NCHUNK, g_body, 0)

        pltpu.sync_copy(cont_hbm.at[pl.ds(base_b * _NUM_CONT, _ROWS)], feat_v)

        def cont_idx_body(j, _):
            pos = j * 16 + lax.iota(jnp.int32, 16)
            f = lax.rem(pos, _NUM_CONT)
            b = lax.div(pos, _NUM_CONT)
            x = feat_v[pl.ds(j * 16, 16)]
            t = jnp.clip((x * jnp.float32(_NUM_BINS)).astype(jnp.int32),
                         0, _NUM_BINS - 1)
            idx_v[pl.ds(j * 16, 16)] = t + f * _NUM_BINS
            didx_v[j // 8, pl.ds((j % 8) * 16, 16)] = (base_b + b) * _NUM_TOK + f
            return 0

        lax.fori_loop(0, _NVEC, cont_idx_body, 0)
        move_rows(bin_hbm)

        for c in range(2):
            cb = base_b + c * _CAT_HALF
            pltpu.sync_copy(cat_hbm.at[pl.ds(cb * _NUM_CAT, _ROWS)], cati_v)

            def cat_idx_body(j, _):
                pos = j * 16 + lax.iota(jnp.int32, 16)
                f = lax.rem(pos, _NUM_CAT)
                b = lax.div(pos, _NUM_CAT)
                v = cati_v[pl.ds(j * 16, 16)]
                idx_v[pl.ds(j * 16, 16)] = v + f * _VOCAB
                didx_v[j // 8, pl.ds((j % 8) * 16, 16)] = (
                    (cb + b) * _NUM_TOK + _NUM_CONT + f
                )
                return 0

            lax.fori_loop(0, _NVEC, cat_idx_body, 0)
            move_rows(ctab_hbm)

    return gather_kernel(cont_flat, cat_flat, bin_tab, cat_tab)


_RB = 6144


def _project(tokens, W, b2):
    def body(tok_ref, w_ref, b_ref, out_ref):
        y = lax.dot_general(tok_ref[...], w_ref[...], (((1,), (1,)), ((), ())),
                            preferred_element_type=jnp.float32)
        out_ref[...] = y + b_ref[...]

    n_rows = _NUM_TOK * _BATCH
    return pl.pallas_call(
        body,
        grid=(n_rows // _RB,),
        in_specs=[
            pl.BlockSpec((_RB, _EMB), lambda i: (i, 0)),
            pl.BlockSpec((_EMB, _EMB), lambda i: (0, 0)),
            pl.BlockSpec((1, _EMB), lambda i: (0, 0)),
        ],
        out_specs=pl.BlockSpec((_RB, _EMB), lambda i: (i, 0)),
        out_shape=jax.ShapeDtypeStruct((n_rows, _EMB), jnp.float32),
    )(tokens, W, b2)


def kernel(continuous_features, categorical_features, bin_tables, cat_tables, W, b):
    tokens = _sc_gather_tokens(
        continuous_features.reshape(-1),
        categorical_features.reshape(-1).astype(jnp.int32),
        bin_tables.reshape(_NUM_CONT * _NUM_BINS, _EMB),
        cat_tables.reshape(_NUM_CAT * _VOCAB, _EMB),
    )
    out = _project(tokens, W, b.reshape(1, _EMB))
    return out.reshape(_BATCH, _NUM_TOK, _EMB)


# speedup vs baseline: 1.0573x; 1.0573x over previous
import functools

import jax
import jax.numpy as jnp
from jax import lax
from jax.experimental import pallas as pl
from jax.experimental.pallas import tpu as pltpu
from jax.experimental.pallas import tpu_sc as plsc

_NUM_CONT = 13
_NUM_CAT = 26
_NUM_TOK = _NUM_CONT + _NUM_CAT
_NUM_BINS = 50
_VOCAB = 100000
_EMB = 64
_BATCH = 4096

_NC = 2
_NS = 16
_NW = _NC * _NS
_BPW = _BATCH // _NW
_ROWS = _BPW * _NUM_CONT
_CAT_HALF = _BPW // 2
_GCHUNK = 128
_NCHUNK = _ROWS // _GCHUNK


def _sc_gather_tokens(cont_t, cat_t, bin_tab, cat_tab):
    mesh = plsc.VectorSubcoreMesh(core_axis_name="c", subcore_axis_name="s")

    @functools.partial(
        pl.kernel,
        out_type=jax.ShapeDtypeStruct((_NUM_TOK * _BATCH, _EMB), jnp.float32),
        mesh=mesh,
        scratch_types=[
            pltpu.VMEM((_NUM_CONT, _BPW), jnp.float32),
            pltpu.VMEM((_NUM_CAT, _CAT_HALF), jnp.int32),
            pltpu.VMEM((_ROWS,), jnp.int32),
            pltpu.VMEM((_ROWS, _EMB), jnp.float32),
            pltpu.SemaphoreType.DMA,
        ],
        compiler_params=pltpu.CompilerParams(
            needs_layout_passes=False, use_tc_tiling_on_sc=False
        ),
    )
    def gather_kernel(cont_hbm, cat_hbm, bin_hbm, ctab_hbm, out_hbm,
                      cv, qv, idx_v, rows_v, sem):
        wid = lax.axis_index("s") * _NC + lax.axis_index("c")
        base_b = wid * _BPW

        def gather_rows(table_hbm):
            def g_body(j, _):
                pltpu.async_copy(
                    table_hbm.at[idx_v.at[pl.ds(j * _GCHUNK, _GCHUNK)]],
                    rows_v.at[pl.ds(j * _GCHUNK, _GCHUNK)],
                    sem,
                ).wait()
                return 0

            lax.fori_loop(0, _NCHUNK, g_body, 0)

        pltpu.sync_copy(cont_hbm.at[:, pl.ds(base_b, _BPW)], cv)
        for f in range(_NUM_CONT):
            def cont_idx_body(j, _, f=f):
                x = cv[f, pl.ds(j * 16, 16)]
                t = jnp.clip((x * jnp.float32(_NUM_BINS)).astype(jnp.int32),
                             0, _NUM_BINS - 1)
                idx_v[pl.ds(f * _BPW + j * 16, 16)] = t + f * _NUM_BINS
                return 0

            lax.fori_loop(0, _BPW // 16, cont_idx_body, 0)
        gather_rows(bin_hbm)

        def cont_out_body(f, _):
            pltpu.sync_copy(
                rows_v.at[pl.ds(f * _BPW, _BPW)],
                out_hbm.at[pl.ds(f * _BATCH + base_b, _BPW)],
            )
            return 0

        lax.fori_loop(0, _NUM_CONT, cont_out_body, 0)

        for c in range(2):
            cb = base_b + c * _CAT_HALF
            pltpu.sync_copy(cat_hbm.at[:, pl.ds(cb, _CAT_HALF)], qv)
            for f in range(_NUM_CAT):
                def cat_idx_body(j, _, f=f):
                    v = qv[f, pl.ds(j * 16, 16)]
                    idx_v[pl.ds(f * _CAT_HALF + j * 16, 16)] = v + f * _VOCAB
                    return 0

                lax.fori_loop(0, _CAT_HALF // 16, cat_idx_body, 0)
            gather_rows(ctab_hbm)

            def cat_out_body(f, _, cb=cb):
                pltpu.sync_copy(
                    rows_v.at[pl.ds(f * _CAT_HALF, _CAT_HALF)],
                    out_hbm.at[pl.ds((_NUM_CONT + f) * _BATCH + cb, _CAT_HALF)],
                )
                return 0

            lax.fori_loop(0, _NUM_CAT, cat_out_body, 0)

    return gather_kernel(cont_t, cat_t, bin_tab, cat_tab)


def _project(tokens, W, b2):
    def body(tok_ref, w_ref, b_ref, out_ref):
        y = lax.dot_general(w_ref[...], tok_ref[...], (((1,), (1,)), ((), ())),
                            preferred_element_type=jnp.float32)
        out_ref[...] = (y + b_ref[...]).reshape(1, _EMB, _BATCH)

    return pl.pallas_call(
        body,
        grid=(_NUM_TOK,),
        in_specs=[
            pl.BlockSpec((_BATCH, _EMB), lambda f: (f, 0)),
            pl.BlockSpec((_EMB, _EMB), lambda f: (0, 0)),
            pl.BlockSpec((_EMB, 1), lambda f: (0, 0)),
        ],
        out_specs=pl.BlockSpec((1, _EMB, _BATCH), lambda f: (f, 0, 0)),
        out_shape=jax.ShapeDtypeStruct((_NUM_TOK, _EMB, _BATCH), jnp.float32),
    )(tokens, W, b2)


def kernel(continuous_features, categorical_features, bin_tables, cat_tables, W, b):
    tokens = _sc_gather_tokens(
        continuous_features.T,
        categorical_features.T.astype(jnp.int32),
        bin_tables.reshape(_NUM_CONT * _NUM_BINS, _EMB),
        cat_tables.reshape(_NUM_CAT * _VOCAB, _EMB),
    )
    out_t = _project(tokens, W, b.reshape(_EMB, 1))
    return jnp.transpose(out_t, (2, 0, 1))
